# baseline (speedup 1.0000x reference)
import functools

import jax
import jax.numpy as jnp
from jax import lax
from jax.experimental import pallas as pl
from jax.experimental.pallas import tpu as pltpu
from jax.experimental.pallas import tpu_sc as plsc

_LANES = 16
_NDEP = 8
_D = 256
_W = 128
_MAXNORM = 1.0 - 1e-5
_MARGIN = 0.001


def _newton_sqrt(s):
    i = lax.bitcast_convert_type(s, jnp.int32)
    i = jnp.int32(0x5F3759DF) - lax.shift_right_arithmetic(i, 1)
    y = lax.bitcast_convert_type(i, jnp.float32)
    for _ in range(4):
        y = y * (1.5 - 0.5 * s * y * y)
    return s * y


@functools.partial(
    pl.kernel,
    out_type=jax.ShapeDtypeStruct((_LANES,), jnp.float32),
    mesh=plsc.VectorSubcoreMesh(core_axis_name="c", subcore_axis_name="s",
                                num_cores=1),
    scratch_types=[
        pltpu.VMEM((_LANES,), jnp.int32),
        pltpu.VMEM((_LANES, _D), jnp.float32),
        pltpu.VMEM((_W,), jnp.float32),
        pltpu.VMEM((_NDEP, _W), jnp.float32),
        pltpu.VMEM_SHARED((_NDEP, _W), jnp.float32),
        pltpu.SemaphoreType.DMA,
    ],
    compiler_params=pltpu.CompilerParams(
        needs_layout_passes=False,
        skip_device_barrier=True,
        disable_bounds_checks=True,
        disable_semaphore_checks=True,
    ),
)
def _dqml_sc(emb_hbm, idx_hbm, out_hbm, idx_v, rows_v, res_v, comb_v,
             shared, sem):
    c = lax.axis_index("c")
    s = lax.axis_index("s")
    lanes = lax.iota(jnp.int32, _LANES)

    @pl.when(jnp.logical_and(c == 0, s < _NDEP))
    def _worker():
        d = s
        pltpu.sync_copy(idx_hbm.at[d], idx_v)
        pltpu.async_copy(emb_hbm.at[idx_v], rows_v, sem).wait()

        def body(j, acc):
            cols = (jnp.broadcast_to(j, (_LANES,)) + lanes) & (_D - 1)
            v = plsc.load_gather(rows_v, [lanes, cols])
            return acc + v * v
        ssq = lax.fori_loop(0, _D, body, jnp.zeros((_LANES,), jnp.float32))

        radii = jnp.minimum(_newton_sqrt(ssq), _MAXNORM)
        srt = lax.sort(radii)
        hi2 = jnp.sum(jnp.where((lanes == 13) | (lanes == 14), srt, 0.0))
        lo2 = jnp.sum(jnp.where((lanes == 1) | (lanes == 2), srt, 0.0))
        res = jnp.where(
            lanes == 0,
            jnp.broadcast_to(hi2, (_LANES,)) * 0.5,
            jnp.where(lanes == 1, jnp.broadcast_to(lo2, (_LANES,)) * 0.5,
                      0.0),
        )
        res_v[pl.ds(0, _LANES)] = res
        pltpu.sync_copy(res_v, shared.at[d])

    plsc.subcore_barrier()

    @pl.when(jnp.logical_and(c == 0, s == 0))
    def _combine():
        pltpu.sync_copy(shared, comb_v)
        highs = plsc.load_gather(
            comb_v, [lanes & (_NDEP - 1), jnp.zeros((_LANES,), jnp.int32)])
        lows_next = plsc.load_gather(
            comb_v, [jnp.minimum(lanes + 1, _NDEP - 1),
                     jnp.ones((_LANES,), jnp.int32)])
        lossv = jnp.maximum(highs + _MARGIN - lows_next, 0.0)
        lossv = jnp.where(lanes < _NDEP - 1, lossv, 0.0)
        tot = jnp.sum(lossv)
        res_v[pl.ds(0, _LANES)] = jnp.broadcast_to(tot, (_LANES,)) / (
            _NDEP - 1.0)
        pltpu.sync_copy(res_v.at[pl.ds(0, _LANES)], out_hbm)


def kernel(embeddings, depth_indices):
    out = _dqml_sc(embeddings, depth_indices)
    return out[0]


# speedup vs baseline: 1.0258x; 1.0258x over previous
import functools

import jax
import jax.numpy as jnp
from jax import lax
from jax.experimental import pallas as pl
from jax.experimental.pallas import tpu as pltpu
from jax.experimental.pallas import tpu_sc as plsc

_LANES = 16
_NDEP = 8
_HALF = 8
_D = 256
_W = 128
_MAXNORM = 1.0 - 1e-5
_MARGIN = 0.001


def _newton_sqrt(s):
    i = lax.bitcast_convert_type(s, jnp.int32)
    i = jnp.int32(0x5F3759DF) - lax.shift_right_arithmetic(i, 1)
    y = lax.bitcast_convert_type(i, jnp.float32)
    for _ in range(4):
        y = y * (1.5 - 0.5 * s * y * y)
    return s * y


@functools.partial(
    pl.kernel,
    out_type=jax.ShapeDtypeStruct((_LANES,), jnp.float32),
    mesh=plsc.VectorSubcoreMesh(core_axis_name="c", subcore_axis_name="s",
                                num_cores=1),
    scratch_types=[
        pltpu.VMEM((_HALF,), jnp.int32),
        pltpu.VMEM((_HALF, _D), jnp.float32),
        pltpu.VMEM((_W,), jnp.float32),
        pltpu.VMEM((_NDEP, _W), jnp.float32),
        pltpu.VMEM_SHARED((_NDEP, _W), jnp.float32),
        pltpu.VMEM_SHARED((_NDEP, _W), jnp.float32),
        pltpu.SemaphoreType.DMA,
    ],
    compiler_params=pltpu.CompilerParams(needs_layout_passes=False),
)
def _dqml_sc(emb_hbm, idx_hbm, out_hbm, idx_v, rows_v, res_v, comb_v,
             shared, shared_b, sem):
    s = lax.axis_index("s")
    d = s & (_NDEP - 1)
    h = lax.shift_right_logical(s, 3)
    lanes = lax.iota(jnp.int32, _LANES)

    pltpu.sync_copy(idx_hbm.at[d, pl.ds(h * _HALF, _HALF)], idx_v)
    pltpu.async_copy(emb_hbm.at[idx_v], rows_v, sem).wait()

    ssq = jnp.zeros((_LANES,), jnp.float32)
    for r in range(_HALF):
        acc = jnp.zeros((_LANES,), jnp.float32)
        for j in range(_D // _LANES):
            v = rows_v[r, pl.ds(j * _LANES, _LANES)]
            acc = acc + v * v
        sr = jnp.sum(acc)
        ssq = jnp.where(lanes == r, jnp.broadcast_to(sr, (_LANES,)), ssq)

    radii = jnp.minimum(_newton_sqrt(ssq), _MAXNORM)
    res_v[pl.ds(0, _LANES)] = radii

    @pl.when(h == 1)
    def _publish_half():
        pltpu.sync_copy(res_v.at[pl.ds(0, _HALF)],
                        shared_b.at[d, pl.ds(0, _HALF)])

    plsc.subcore_barrier()

    @pl.when(h == 0)
    def _sort_depth():
        pltpu.sync_copy(shared_b.at[d, pl.ds(0, _HALF)],
                        res_v.at[pl.ds(_HALF, _HALF)])
        srt = lax.sort(res_v[pl.ds(0, _LANES)])
        hi2 = jnp.sum(jnp.where((lanes == 13) | (lanes == 14), srt, 0.0))
        lo2 = jnp.sum(jnp.where((lanes == 1) | (lanes == 2), srt, 0.0))
        res = jnp.where(
            lanes == 0,
            jnp.broadcast_to(hi2, (_LANES,)) * 0.5,
            jnp.where(lanes == 1, jnp.broadcast_to(lo2, (_LANES,)) * 0.5,
                      0.0),
        )
        res_v[pl.ds(0, _LANES)] = res
        pltpu.sync_copy(res_v.at[pl.ds(0, _LANES)],
                        shared.at[d, pl.ds(0, _LANES)])

    plsc.subcore_barrier()

    @pl.when(s == 0)
    def _combine():
        pltpu.sync_copy(shared, comb_v)
        highs = plsc.load_gather(
            comb_v, [lanes & (_NDEP - 1), jnp.zeros((_LANES,), jnp.int32)])
        lows_next = plsc.load_gather(
            comb_v, [jnp.minimum(lanes + 1, _NDEP - 1),
                     jnp.ones((_LANES,), jnp.int32)])
        lossv = jnp.maximum(highs + _MARGIN - lows_next, 0.0)
        lossv = jnp.where(lanes < _NDEP - 1, lossv, 0.0)
        tot = jnp.sum(lossv)
        res_v[pl.ds(0, _LANES)] = jnp.broadcast_to(tot, (_LANES,)) / (
            _NDEP - 1.0)
        pltpu.sync_copy(res_v.at[pl.ds(0, _LANES)], out_hbm)


def kernel(embeddings, depth_indices):
    out = _dqml_sc(embeddings, depth_indices)
    return out[0]
